# Initial kernel scaffold; baseline (speedup 1.0000x reference)
import jax
import jax.numpy as jnp
from jax.experimental import pallas as pl


def kernel(x, edge_index, W_l1, b_l1, W_r1, W_l2, b_l2, W_r2):
    raise NotImplementedError("write your pallas kernel here")


# speedup vs baseline: 3.7839x; 3.7839x over previous
import functools

import jax
import jax.numpy as jnp
from jax import lax
from jax.experimental import pallas as pl
from jax.experimental.pallas import tpu as pltpu
from jax.experimental.pallas import tpu_sc as plsc

N_NODES = 10000
D = 128
N_EDGES = 320000

NC = 2
NS = 16
CH = 128
EDGES_PER_W = -(-N_EDGES // (NC * NS * CH)) * CH
E_PAD = EDGES_PER_W * NC * NS
CPW = EDGES_PER_W // CH

ACC_ROWS = 10240
RPS = ACC_ROWS // NS

BR = 1000


def _sc_aggregate(feats, src1d, dst1d, zeros_d):
    mesh = plsc.VectorSubcoreMesh(core_axis_name="c", subcore_axis_name="s")

    @functools.partial(
        pl.kernel, mesh=mesh,
        out_type=[jax.ShapeDtypeStruct((NC * ACC_ROWS, D), jnp.float32)],
        scratch_types=[
            pltpu.VMEM((CH,), jnp.int32),
            pltpu.VMEM((CH,), jnp.int32),
            pltpu.VMEM((CH, D), jnp.float32),
            pltpu.VMEM_SHARED((ACC_ROWS, D), jnp.float32),
            pltpu.SemaphoreType.DMA,
        ])
    def k(feats_hbm, src_hbm, dst_hbm, z_hbm, agg_hbm,
          src_v, dst_v, msg_v, acc_sh, sem):
        c = lax.axis_index("c")
        s = lax.axis_index("s")

        pltpu.sync_copy(z_hbm, acc_sh.at[pl.ds(s * RPS, RPS)])
        plsc.subcore_barrier()

        base_e = (c * NS + s) * EDGES_PER_W

        @pl.loop(0, CPW)
        def _(i):
            off = base_e + i * CH
            pltpu.sync_copy(src_hbm.at[pl.ds(off, CH)], src_v)
            pltpu.sync_copy(dst_hbm.at[pl.ds(off, CH)], dst_v)
            pltpu.async_copy(feats_hbm.at[src_v], msg_v, sem).wait()
            pltpu.sync_copy(msg_v, acc_sh.at[dst_v], add=True)

        plsc.subcore_barrier()
        pltpu.sync_copy(acc_sh.at[pl.ds(s * RPS, RPS)],
                        agg_hbm.at[pl.ds(c * ACC_ROWS + s * RPS, RPS)])

    return k(feats, src1d, dst1d, zeros_d)[0]


def _sc_count(dst1d, zeros_d, ones_d):
    mesh = plsc.VectorSubcoreMesh(core_axis_name="c", subcore_axis_name="s")

    @functools.partial(
        pl.kernel, mesh=mesh,
        out_type=[jax.ShapeDtypeStruct((NC * ACC_ROWS, D), jnp.float32)],
        scratch_types=[
            pltpu.VMEM((CH,), jnp.int32),
            pltpu.VMEM((CH, D), jnp.float32),
            pltpu.VMEM_SHARED((ACC_ROWS, D), jnp.float32),
        ])
    def k(dst_hbm, z_hbm, ones_hbm, cnt_hbm, dst_v, ones_v, acc_sh):
        c = lax.axis_index("c")
        s = lax.axis_index("s")

        pltpu.sync_copy(z_hbm, acc_sh.at[pl.ds(s * RPS, RPS)])
        pltpu.sync_copy(ones_hbm, ones_v)
        plsc.subcore_barrier()

        base_e = (c * NS + s) * EDGES_PER_W

        @pl.loop(0, CPW)
        def _(i):
            off = base_e + i * CH
            pltpu.sync_copy(dst_hbm.at[pl.ds(off, CH)], dst_v)
            pltpu.sync_copy(ones_v, acc_sh.at[dst_v], add=True)

        plsc.subcore_barrier()
        pltpu.sync_copy(acc_sh.at[pl.ds(s * RPS, RPS)],
                        cnt_hbm.at[pl.ds(c * ACC_ROWS + s * RPS, RPS)])

    return k(dst1d, zeros_d, ones_d)[0]


def _tc_layer1(agg, cnt, x, W_l, b, W_r):
    def body(agg_ref, cnt_ref, x_ref, wl_ref, b_ref, wr_ref, h_ref, rcp_ref):
        a = agg_ref[0] + agg_ref[1]
        cn = cnt_ref[0] + cnt_ref[1]
        rb = 1.0 / jnp.maximum(cn, 1.0)
        mean = a * rb
        h = (jnp.dot(mean, wl_ref[...], preferred_element_type=jnp.float32)
             + b_ref[...]
             + jnp.dot(x_ref[...], wr_ref[...],
                       preferred_element_type=jnp.float32))
        h_ref[...] = jnp.maximum(h, 0.0)
        rcp_ref[...] = rb

    grid = (N_NODES // BR,)
    return pl.pallas_call(
        body,
        grid=grid,
        in_specs=[
            pl.BlockSpec((NC, BR, D), lambda i: (0, i, 0)),
            pl.BlockSpec((NC, BR, D), lambda i: (0, i, 0)),
            pl.BlockSpec((BR, D), lambda i: (i, 0)),
            pl.BlockSpec((D, D), lambda i: (0, 0)),
            pl.BlockSpec((1, D), lambda i: (0, 0)),
            pl.BlockSpec((D, D), lambda i: (0, 0)),
        ],
        out_specs=[
            pl.BlockSpec((BR, D), lambda i: (i, 0)),
            pl.BlockSpec((BR, D), lambda i: (i, 0)),
        ],
        out_shape=[
            jax.ShapeDtypeStruct((N_NODES, D), jnp.float32),
            jax.ShapeDtypeStruct((N_NODES, D), jnp.float32),
        ],
    )(agg, cnt, x, W_l, b, W_r)


def _tc_layer2(agg, rcp, h, W_l, b, W_r):
    def body(agg_ref, rcp_ref, h_ref, wl_ref, b_ref, wr_ref, out_ref):
        a = agg_ref[0] + agg_ref[1]
        mean = a * rcp_ref[...]
        out_ref[...] = (
            jnp.dot(mean, wl_ref[...], preferred_element_type=jnp.float32)
            + b_ref[...]
            + jnp.dot(h_ref[...], wr_ref[...],
                      preferred_element_type=jnp.float32))

    grid = (N_NODES // BR,)
    return pl.pallas_call(
        body,
        grid=grid,
        in_specs=[
            pl.BlockSpec((NC, BR, D), lambda i: (0, i, 0)),
            pl.BlockSpec((BR, D), lambda i: (i, 0)),
            pl.BlockSpec((BR, D), lambda i: (i, 0)),
            pl.BlockSpec((D, D), lambda i: (0, 0)),
            pl.BlockSpec((1, D), lambda i: (0, 0)),
            pl.BlockSpec((D, D), lambda i: (0, 0)),
        ],
        out_specs=pl.BlockSpec((BR, D), lambda i: (i, 0)),
        out_shape=jax.ShapeDtypeStruct((N_NODES, D), jnp.float32),
    )(agg, rcp, h, W_l, b, W_r)


def kernel(x, edge_index, W_l1, b_l1, W_r1, W_l2, b_l2, W_r2):
    src = edge_index[0]
    dst = edge_index[1]
    pad = E_PAD - N_EDGES
    src1d = jnp.concatenate([src, jnp.zeros((pad,), jnp.int32)])
    dst1d = jnp.concatenate([dst, jnp.full((pad,), N_NODES, jnp.int32)])

    zeros_d = jnp.zeros((RPS, D), jnp.float32)
    ones_d = jnp.ones((CH, D), jnp.float32)

    cnt = _sc_count(dst1d, zeros_d, ones_d).reshape(NC, ACC_ROWS, D)
    agg1 = _sc_aggregate(x, src1d, dst1d, zeros_d).reshape(NC, ACC_ROWS, D)
    h, rcp = _tc_layer1(agg1, cnt, x, W_l1, b_l1.reshape(1, D), W_r1)

    agg2 = _sc_aggregate(h, src1d, dst1d, zeros_d).reshape(NC, ACC_ROWS, D)
    out = _tc_layer2(agg2, rcp, h, W_l2, b_l2.reshape(1, D), W_r2)
    return out
